# baseline (speedup 1.0000x reference)
import jax
import jax.numpy as jnp
from jax import lax
from jax.experimental import pallas as pl
from jax.experimental.pallas import tpu as pltpu
from jax.experimental.pallas import tpu_sc as plsc

NUM_TOKENS = 4096
SEQ = 200
DIM = 32
NUM_ROWS = 1000000
LANE = 16

NC = 2
NS = 16
NW = NC * NS

TCOLS = 7813
PAD_ROWS = TCOLS * 128
COLS_MAX = 245
A_NBUF = 4


def _relayout_body(tt_hbm, flat_hbm, vi0, vi1, vi2, vi3, vo0, vo1, vo2, vo3,
                   seml, sems):
    wid = lax.axis_index("s") * NC + lax.axis_index("c")
    n_cols = 244 + jnp.where(wid < 5, 1, 0)
    base = wid * 244 + jnp.minimum(wid, 5)
    vins = (vi0, vi1, vi2, vi3)
    vouts = (vo0, vo1, vo2, vo3)

    def start_load(i, b):
        col = base + i
        for c4 in range(4):
            pltpu.async_copy(
                tt_hbm.at[pl.ds(8 * c4, 8), pl.ds(col * 128, 128)],
                vins[b].at[pl.ds(8 * c4, 8)],
                seml.at[b],
            )

    def wait_load(b):
        pltpu.make_async_copy(
            tt_hbm.at[:, pl.ds(0, 128)], vins[b], seml.at[b]
        ).wait()

    def start_store(i, b):
        col = base + i
        pltpu.async_copy(
            vouts[b], flat_hbm.at[pl.ds(col * 4096, 4096)], sems.at[b]
        )

    def wait_store(b):
        pltpu.make_async_copy(
            vouts[b], flat_hbm.at[pl.ds(0, 4096)], sems.at[b]
        ).wait()

    def transpose_block(b):
        for lb in range(8):
            sv = (jnp.arange(LANE, dtype=jnp.int32) + (16 * lb)) * DIM
            for c in range(32):
                x = vins[b][c, pl.ds(16 * lb, LANE)]
                plsc.store_scatter(vouts[b], [sv + c], x)

    for b in range(A_NBUF):
        start_load(b, b)

    def turn(g, carry):
        for b in range(A_NBUF):
            i = A_NBUF * g + b
            p = i < n_cols

            @pl.when(p)
            def _():
                wait_load(b)

            @pl.when(jnp.logical_and(p, i >= A_NBUF))
            def _():
                wait_store(b)

            @pl.when(p)
            def _():
                transpose_block(b)
                start_store(i, b)

            @pl.when((i + A_NBUF) < n_cols)
            def _():
                start_load(i + A_NBUF, b)

        return carry

    lax.fori_loop(0, (COLS_MAX + A_NBUF - 1) // A_NBUF, turn, 0)
    for b in range(A_NBUF):
        wait_store(b)


@jax.jit
def _relayout(tt):
    mesh = plsc.VectorSubcoreMesh(core_axis_name="c", subcore_axis_name="s")
    return pl.kernel(
        _relayout_body,
        out_type=jax.ShapeDtypeStruct((PAD_ROWS * DIM,), jnp.float32),
        mesh=mesh,
        scratch_types=(
            [pltpu.VMEM((32, 128), jnp.float32)] * A_NBUF
            + [pltpu.VMEM((4096,), jnp.float32)] * A_NBUF
            + [
                pltpu.SemaphoreType.DMA((A_NBUF,)),
                pltpu.SemaphoreType.DMA((A_NBUF,)),
            ]
        ),
        compiler_params=pltpu.CompilerParams(
            use_tc_tiling_on_sc=True, needs_layout_passes=False
        ),
    )(tt)


ROWS_PER_W = NUM_TOKENS // NW
NBUF = 8
NGROUP = ROWS_PER_W // NBUF


def _gather_body(idx_hbm, table_hbm, out_hbm, idx_all, rows, semg, sems):
    wid = lax.axis_index("s") * NC + lax.axis_index("c")
    r0 = wid * ROWS_PER_W

    pltpu.sync_copy(idx_hbm.at[pl.ds(r0, ROWS_PER_W)], idx_all)

    def start_gather(i, b):
        pltpu.async_copy(table_hbm.at[idx_all.at[i]], rows.at[b], semg.at[b])

    def wait_gather(b):
        pltpu.make_async_copy(
            table_hbm.at[pl.ds(0, SEQ)], rows.at[b], semg.at[b]
        ).wait()

    def start_store(i, b):
        pltpu.async_copy(rows.at[b], out_hbm.at[r0 + i], sems.at[b])

    def wait_store(b):
        pltpu.make_async_copy(rows.at[b], out_hbm.at[0], sems.at[b]).wait()

    for b in range(NBUF):
        start_gather(b, b)

    def turn(g, carry):
        i0 = g * NBUF
        for b in range(NBUF):
            wait_gather(b)
            start_store(i0 + b, b)
        for b in range(NBUF):
            wait_store(b)
            start_gather(i0 + NBUF + b, b)
        return carry

    lax.fori_loop(0, NGROUP - 1, turn, 0)

    i0 = (NGROUP - 1) * NBUF
    for b in range(NBUF):
        wait_gather(b)
        start_store(i0 + b, b)
    for b in range(NBUF):
        wait_store(b)


@jax.jit
def _embed(token_ids, table):
    mesh = plsc.VectorSubcoreMesh(core_axis_name="c", subcore_axis_name="s")
    return pl.kernel(
        _gather_body,
        out_type=jax.ShapeDtypeStruct((NUM_TOKENS, SEQ, DIM), jnp.float32),
        mesh=mesh,
        scratch_types=[
            pltpu.VMEM((ROWS_PER_W, SEQ), jnp.int32),
            pltpu.VMEM((NBUF, SEQ, DIM), jnp.float32),
            pltpu.SemaphoreType.DMA((NBUF,)),
            pltpu.SemaphoreType.DMA((NBUF,)),
        ],
        compiler_params=pltpu.CompilerParams(use_tc_tiling_on_sc=False),
    )(token_ids, table)


def kernel(token_ids, embeddings):
    flat = _relayout(embeddings.T)
    table = flat.reshape(PAD_ROWS, DIM)
    return _embed(jnp.asarray(token_ids, jnp.int32), table)


# speedup vs baseline: 1.1947x; 1.1947x over previous
import jax
import jax.numpy as jnp
from jax import lax
from jax.experimental import pallas as pl
from jax.experimental.pallas import tpu as pltpu
from jax.experimental.pallas import tpu_sc as plsc

NUM_TOKENS = 4096
SEQ = 200
DIM = 32
NUM_ROWS = 1000000
LANE = 16

NC = 2
NS = 16
NW = NC * NS

TCOLS = 7813
PAD_ROWS = TCOLS * 128
COLS_MAX = 245
A_NBUF = 4


def _relayout_body(tt_hbm, flat_hbm, vi0, vi1, vi2, vi3, vo0, vo1, vo2, vo3,
                   seml, sems):
    wid = lax.axis_index("s") * NC + lax.axis_index("c")
    n_cols = 244 + jnp.where(wid < 5, 1, 0)
    base = wid * 244 + jnp.minimum(wid, 5)
    vins = (vi0, vi1, vi2, vi3)
    vouts = (vo0, vo1, vo2, vo3)

    def start_load(i, b):
        col = base + i
        for c4 in range(4):
            pltpu.async_copy(
                tt_hbm.at[pl.ds(8 * c4, 8), pl.ds(col * 128, 128)],
                vins[b].at[pl.ds(8 * c4, 8)],
                seml.at[b],
            )

    def wait_load(b):
        pltpu.make_async_copy(
            tt_hbm.at[:, pl.ds(0, 128)], vins[b], seml.at[b]
        ).wait()

    def start_store(i, b):
        col = base + i
        pltpu.async_copy(
            vouts[b], flat_hbm.at[pl.ds(col * 4096, 4096)], sems.at[b]
        )

    def wait_store(b):
        pltpu.make_async_copy(
            vouts[b], flat_hbm.at[pl.ds(0, 4096)], sems.at[b]
        ).wait()

    def transpose_block(b):
        vin, vout = vins[b], vouts[b]

        @plsc.parallel_loop(0, 32, unroll=8)
        def _(c):
            for lb in range(8):
                sv = (jnp.arange(LANE, dtype=jnp.int32) + (16 * lb)) * DIM
                x = vin[c, pl.ds(16 * lb, LANE)]
                plsc.store_scatter(vout, [sv + c], x)

    for b in range(A_NBUF):
        start_load(b, b)

    def turn(g, carry):
        for b in range(A_NBUF):
            i = A_NBUF * g + b
            p = i < n_cols

            @pl.when(p)
            def _():
                wait_load(b)

            @pl.when(jnp.logical_and(p, i >= A_NBUF))
            def _():
                wait_store(b)

            @pl.when(p)
            def _():
                transpose_block(b)
                start_store(i, b)

            @pl.when((i + A_NBUF) < n_cols)
            def _():
                start_load(i + A_NBUF, b)

        return carry

    lax.fori_loop(0, (COLS_MAX + A_NBUF - 1) // A_NBUF, turn, 0)
    for b in range(A_NBUF):
        wait_store(b)


@jax.jit
def _relayout(tt):
    mesh = plsc.VectorSubcoreMesh(core_axis_name="c", subcore_axis_name="s")
    return pl.kernel(
        _relayout_body,
        out_type=jax.ShapeDtypeStruct((PAD_ROWS * DIM,), jnp.float32),
        mesh=mesh,
        scratch_types=(
            [pltpu.VMEM((32, 128), jnp.float32)] * A_NBUF
            + [pltpu.VMEM((4096,), jnp.float32)] * A_NBUF
            + [
                pltpu.SemaphoreType.DMA((A_NBUF,)),
                pltpu.SemaphoreType.DMA((A_NBUF,)),
            ]
        ),
        compiler_params=pltpu.CompilerParams(
            use_tc_tiling_on_sc=True, needs_layout_passes=False
        ),
    )(tt)


ROWS_PER_W = NUM_TOKENS // NW
NBUF = 8
NGROUP = ROWS_PER_W // NBUF


def _gather_body(idx_hbm, table_hbm, out_hbm, idx_all, rows, semg, sems):
    wid = lax.axis_index("s") * NC + lax.axis_index("c")
    r0 = wid * ROWS_PER_W

    pltpu.sync_copy(idx_hbm.at[pl.ds(r0, ROWS_PER_W)], idx_all)

    def start_gather(i, b):
        pltpu.async_copy(table_hbm.at[idx_all.at[i]], rows.at[b], semg.at[b])

    def wait_gather(b):
        pltpu.make_async_copy(
            table_hbm.at[pl.ds(0, SEQ)], rows.at[b], semg.at[b]
        ).wait()

    def start_store(i, b):
        pltpu.async_copy(rows.at[b], out_hbm.at[r0 + i], sems.at[b])

    def wait_store(b):
        pltpu.make_async_copy(rows.at[b], out_hbm.at[0], sems.at[b]).wait()

    for b in range(NBUF):
        start_gather(b, b)

    def turn(g, carry):
        i0 = g * NBUF
        for b in range(NBUF):
            wait_gather(b)
            start_store(i0 + b, b)
        for b in range(NBUF):
            wait_store(b)
            start_gather(i0 + NBUF + b, b)
        return carry

    lax.fori_loop(0, NGROUP - 1, turn, 0)

    i0 = (NGROUP - 1) * NBUF
    for b in range(NBUF):
        wait_gather(b)
        start_store(i0 + b, b)
    for b in range(NBUF):
        wait_store(b)


@jax.jit
def _embed(token_ids, table):
    mesh = plsc.VectorSubcoreMesh(core_axis_name="c", subcore_axis_name="s")
    return pl.kernel(
        _gather_body,
        out_type=jax.ShapeDtypeStruct((NUM_TOKENS, SEQ, DIM), jnp.float32),
        mesh=mesh,
        scratch_types=[
            pltpu.VMEM((ROWS_PER_W, SEQ), jnp.int32),
            pltpu.VMEM((NBUF, SEQ, DIM), jnp.float32),
            pltpu.SemaphoreType.DMA((NBUF,)),
            pltpu.SemaphoreType.DMA((NBUF,)),
        ],
        compiler_params=pltpu.CompilerParams(use_tc_tiling_on_sc=False),
    )(token_ids, table)


def kernel(token_ids, embeddings):
    flat = _relayout(embeddings.T)
    table = flat.reshape(PAD_ROWS, DIM)
    return _embed(jnp.asarray(token_ids, jnp.int32), table)
